# baseline (speedup 1.0000x reference)
import functools

import jax
import jax.numpy as jnp
from jax import lax
from jax.experimental import pallas as pl
from jax.experimental.pallas import tpu as pltpu
from jax.experimental.pallas import tpu_sc as plsc

D = 128
LANES = 16
N_TILES = 32
WI = 128
WT = WI // 2


def _lookup_add(x2, idx, pe2):
    N = x2.shape[0]
    toks_per_tile = N // N_TILES
    n_windows = toks_per_tile // WT

    mesh = plsc.VectorSubcoreMesh(core_axis_name="c", subcore_axis_name="s")

    @functools.partial(
        pl.kernel,
        out_type=jax.ShapeDtypeStruct((N, D), jnp.float32),
        mesh=mesh,
        scratch_types=[
            pltpu.VMEM((WI,), jnp.int32),
            pltpu.VMEM((WI,), jnp.int32),
            pltpu.VMEM((WI, D), jnp.float32),
            pltpu.VMEM((WI, D), jnp.float32),
            pltpu.VMEM((WT, D), jnp.float32),
            pltpu.VMEM((WT, D), jnp.float32),
            pltpu.SemaphoreType.DMA((2,)),
            pltpu.SemaphoreType.DMA((2,)),
            pltpu.SemaphoreType.DMA((2,)),
            pltpu.SemaphoreType.DMA((2,)),
        ],
    )
    def k(x_hbm, idx_hbm, pe_hbm, out_hbm,
          idx0, idx1, g0, g1, xv0, xv1, isem, gsem, xsem, osem):
        wid = lax.axis_index("s") * 2 + lax.axis_index("c")
        idx_base = wid * toks_per_tile * 2
        tok_base = wid * toks_per_tile
        idx_b = (idx0, idx1)
        g_b = (g0, g1)
        x_b = (xv0, xv1)

        def idx_copy(w, b):
            return pltpu.make_async_copy(
                idx_hbm.at[pl.ds(idx_base + w * WI, WI)], idx_b[b], isem.at[b])

        def gather_copy(w, b):
            del w
            return pltpu.make_async_copy(pe_hbm.at[idx_b[b]], g_b[b], gsem.at[b])

        def x_copy(w, b):
            return pltpu.make_async_copy(
                x_hbm.at[pl.ds(tok_base + w * WT, WT)], x_b[b], xsem.at[b])

        def out_copy(w, b):
            return pltpu.make_async_copy(
                x_b[b], out_hbm.at[pl.ds(tok_base + w * WT, WT)], osem.at[b])

        idx_copy(0, 0).start()
        idx_copy(1, 1).start()
        idx_copy(0, 0).wait()
        gather_copy(0, 0).start()
        x_copy(0, 0).start()

        @pl.loop(0, n_windows // 2)
        def _(h):
            for b in (0, 1):
                w = 2 * h + b
                nb = 1 - b

                @pl.when(w >= 1)
                def _():
                    out_copy(w - 1, nb).wait()

                @pl.when(w + 1 < n_windows)
                def _():
                    idx_copy(w + 1, nb).wait()
                    gather_copy(w + 1, nb).start()
                    x_copy(w + 1, nb).start()

                gather_copy(w, b).wait()
                x_copy(w, b).wait()

                @pl.when(w + 2 < n_windows)
                def _():
                    idx_copy(w + 2, b).start()

                @pl.loop(0, WT, unroll=8)
                def _(t):
                    for j in range(4):
                        s = pl.ds(j * LANES, LANES)
                        plsc.addupdate(x_b[b].at[t, s], g_b[b][2 * t, s])
                    for j in range(4, 8):
                        s = pl.ds(j * LANES, LANES)
                        plsc.addupdate(x_b[b].at[t, s], g_b[b][2 * t + 1, s])

                out_copy(w, b).start()

        out_copy(n_windows - 1, 1).wait()

    return k(x2, idx, pe2)


def kernel(x, positions, pe):
    B, T, _ = x.shape
    N = B * T
    x2 = x.reshape(N, D)
    idx = positions.reshape(2 * N)
    pe2 = jnp.concatenate([pe, pe], axis=1)
    out2 = _lookup_add(x2, idx, pe2)
    return out2.reshape(B, T, D)


# speedup vs baseline: 2.5318x; 2.5318x over previous
import functools

import jax
import jax.numpy as jnp
from jax import lax
from jax.experimental import pallas as pl
from jax.experimental.pallas import tpu as pltpu
from jax.experimental.pallas import tpu_sc as plsc

D = 128
LANES = 16
N_TILES = 32
WT = 128


def _lookup_add(x3, qidx, pe2, B, T):
    KB = B // WT
    n_total = T * KB
    wpt = n_total // N_TILES

    mesh = plsc.VectorSubcoreMesh(core_axis_name="c", subcore_axis_name="s")

    @functools.partial(
        pl.kernel,
        out_type=jax.ShapeDtypeStruct((B, T, D), jnp.float32),
        mesh=mesh,
        scratch_types=[
            pltpu.VMEM((WT,), jnp.int32),
            pltpu.VMEM((WT,), jnp.int32),
            pltpu.VMEM((WT,), jnp.int32),
            pltpu.VMEM((WT,), jnp.int32),
            pltpu.VMEM((WT, D), jnp.float32),
            pltpu.VMEM((WT, D), jnp.float32),
            pltpu.VMEM((WT, D), jnp.float32),
            pltpu.VMEM((WT, D), jnp.float32),
            pltpu.VMEM((WT, D), jnp.float32),
            pltpu.VMEM((WT, D), jnp.float32),
            pltpu.SemaphoreType.DMA((2,)),
            pltpu.SemaphoreType.DMA((2,)),
            pltpu.SemaphoreType.DMA((2,)),
            pltpu.SemaphoreType.DMA((2,)),
            pltpu.SemaphoreType.DMA((2,)),
            pltpu.SemaphoreType.DMA((2,)),
        ],
    )
    def k(x_hbm, idx_hbm, pe_hbm, out_hbm,
          ir0, ir1, ic0, ic1, gr0, gr1, gc0, gc1, xv0, xv1,
          irsem, icsem, rsem, csem, xsem, osem):
        wid = lax.axis_index("s") * 2 + lax.axis_index("c")
        wbase = wid * wpt
        ir_b = (ir0, ir1)
        ic_b = (ic0, ic1)
        gr_b = (gr0, gr1)
        gc_b = (gc0, gc1)
        x_b = (xv0, xv1)

        def ir_copy(w, b):
            g = wbase + w
            return pltpu.make_async_copy(
                idx_hbm.at[pl.ds(g * 2 * WT, WT)], ir_b[b], irsem.at[b])

        def ic_copy(w, b):
            g = wbase + w
            return pltpu.make_async_copy(
                idx_hbm.at[pl.ds(g * 2 * WT + WT, WT)], ic_b[b], icsem.at[b])

        def idx_start(w, b):
            ir_copy(w, b).start()
            ic_copy(w, b).start()

        def idx_wait(w, b):
            ir_copy(w, b).wait()
            ic_copy(w, b).wait()

        def row_gather(w, b):
            del w
            return pltpu.make_async_copy(pe_hbm.at[ir_b[b]], gr_b[b], rsem.at[b])

        def col_gather(w, b):
            del w
            return pltpu.make_async_copy(pe_hbm.at[ic_b[b]], gc_b[b], csem.at[b])

        def x_slice(w):
            g = wbase + w
            t = g // KB
            kk = g - t * KB
            return (pl.ds(kk * WT, WT), t)

        def x_copy(w, b):
            return pltpu.make_async_copy(x_hbm.at[x_slice(w)], x_b[b], xsem.at[b])

        def out_copy(w, b):
            return pltpu.make_async_copy(x_b[b], out_hbm.at[x_slice(w)], osem.at[b])

        idx_start(0, 0)
        idx_start(1, 1)
        idx_wait(0, 0)
        row_gather(0, 0).start()
        col_gather(0, 0).start()
        x_copy(0, 0).start()

        @pl.loop(0, wpt // 2)
        def _(h):
            for b in (0, 1):
                w = 2 * h + b
                nb = 1 - b

                @pl.when(w >= 1)
                def _():
                    out_copy(w - 1, nb).wait()

                @pl.when(w + 1 < wpt)
                def _():
                    idx_wait(w + 1, nb)
                    row_gather(w + 1, nb).start()
                    col_gather(w + 1, nb).start()
                    x_copy(w + 1, nb).start()

                row_gather(w, b).wait()
                col_gather(w, b).wait()
                x_copy(w, b).wait()

                @pl.when(w + 2 < wpt)
                def _():
                    idx_start(w + 2, b)

                @pl.loop(0, WT, unroll=8)
                def _(t):
                    for j in range(4):
                        s = pl.ds(j * LANES, LANES)
                        plsc.addupdate(x_b[b].at[t, s], gr_b[b][t, s])
                    for j in range(4, 8):
                        s = pl.ds(j * LANES, LANES)
                        plsc.addupdate(x_b[b].at[t, s], gc_b[b][t, s])

                out_copy(w, b).start()

        out_copy(wpt - 1, 1).wait()

    return k(x3, qidx, pe2)


def kernel(x, positions, pe):
    B, T, _ = x.shape
    qidx = (positions.transpose(1, 0, 2)
            .reshape(T, B // 128, 128, 2)
            .transpose(0, 1, 3, 2)
            .reshape(-1))
    pe2 = jnp.concatenate([pe, pe], axis=1)
    out = _lookup_add(x, qidx, pe2, B, T)
    return out
